# Initial kernel scaffold; baseline (speedup 1.0000x reference)
import jax
import jax.numpy as jnp
from jax.experimental import pallas as pl


def kernel(z_mask, pos1, pos2, pos3, pos4, gidx1, gidx2, gidx3, gidx4, e1_src, e1_dst, e2_src, e2_dst, e3_src, e3_dst, W1a, b1a, W1b, b1b, W2a, b2a, W2b, b2b, W3a, b3a, W3b, b3b, Wl, bl):
    raise NotImplementedError("write your pallas kernel here")


# speedup vs baseline: 5.0050x; 5.0050x over previous
import functools

import jax
import jax.numpy as jnp
from jax import lax
from jax.experimental import pallas as pl
from jax.experimental.pallas import tpu as pltpu
from jax.experimental.pallas import tpu_sc as plsc

_NW = 32


def _prep_block(x_ref, p_ref, wt_ref, wp_ref, b_ref, o_ref, *, act):
    x = x_ref[...]
    if act:
        x = jnp.where(jnp.isfinite(x), x, 0.0)
        x = jnp.where(x > 0, x, (jnp.exp(x) - 1.0))
    o_ref[...] = (jnp.dot(x, wt_ref[...], preferred_element_type=jnp.float32)
                  + jnp.dot(p_ref[...], wp_ref[...],
                            preferred_element_type=jnp.float32)
                  + b_ref[...])


def _pick_bn(N):
    for bn in (4000, 2000, 1000, 200, 40):
        if N % bn == 0:
            return bn
    return N


def _prep_A(x, pos, Wt, Wp, ba, act):
    N, F = x.shape
    H = Wt.shape[1]
    BN = _pick_bn(N)
    return pl.pallas_call(
        functools.partial(_prep_block, act=act),
        grid=(N // BN,),
        in_specs=[
            pl.BlockSpec((BN, F), lambda i: (i, 0)),
            pl.BlockSpec((BN, 3), lambda i: (i, 0)),
            pl.BlockSpec((F, H), lambda i: (0, 0)),
            pl.BlockSpec((3, H), lambda i: (0, 0)),
            pl.BlockSpec((1, H), lambda i: (0, 0)),
        ],
        out_specs=pl.BlockSpec((BN, H), lambda i: (i, 0)),
        out_shape=jax.ShapeDtypeStruct((N, H), jnp.float32),
    )(x, pos, Wt, Wp, ba.reshape(1, H))


def _prep_B_block(p_ref, wp_ref, o_ref):
    o_ref[...] = jnp.dot(p_ref[...], wp_ref[...],
                         preferred_element_type=jnp.float32)


def _prep_B(pos, Wp):
    N = pos.shape[0]
    H = Wp.shape[1]
    BN = _pick_bn(N)
    return pl.pallas_call(
        _prep_B_block,
        grid=(N // BN,),
        in_specs=[
            pl.BlockSpec((BN, 3), lambda i: (i, 0)),
            pl.BlockSpec((3, H), lambda i: (0, 0)),
        ],
        out_specs=pl.BlockSpec((BN, H), lambda i: (i, 0)),
        out_shape=jax.ShapeDtypeStruct((N, H), jnp.float32),
    )(pos, Wp)


def _edge_mlp_block(g_ref, w_ref, b_ref, o_ref):
    o_ref[...] = (jnp.dot(g_ref[...], w_ref[...],
                          preferred_element_type=jnp.float32) + b_ref[...])


def _edge_mlp(G, Wb, bb, pad_rows, BE=6400):
    E, H = G.shape
    F = Wb.shape[1]
    return pl.pallas_call(
        _edge_mlp_block,
        grid=(E // BE,),
        in_specs=[
            pl.BlockSpec((BE, H), lambda i: (i, 0)),
            pl.BlockSpec((H, F), lambda i: (0, 0)),
            pl.BlockSpec((1, F), lambda i: (0, 0)),
        ],
        out_specs=pl.BlockSpec((BE, F), lambda i: (i, 0)),
        out_shape=jax.ShapeDtypeStruct((E + pad_rows, F), jnp.float32),
    )(G, Wb, bb.reshape(1, F))


def _final_block(x_ref, w_ref, b_ref, o_ref):
    x = x_ref[...]
    x = jnp.where(jnp.isfinite(x), x, 0.0)
    x = jnp.where(x > 0, x, (jnp.exp(x) - 1.0))
    y = jnp.sum(x * w_ref[...], axis=1, keepdims=True) + b_ref[0, 0]
    o_ref[...] = jnp.minimum(y, 0.0) - jnp.log(1.0 + jnp.exp(-jnp.abs(y)))


def _final(x, Wl, bl, BN=25000):
    N, F = x.shape
    return pl.pallas_call(
        _final_block,
        grid=(N // BN,),
        in_specs=[
            pl.BlockSpec((BN, F), lambda i: (i, 0)),
            pl.BlockSpec((1, F), lambda i: (0, 0)),
            pl.BlockSpec((1, 1), lambda i: (0, 0)),
        ],
        out_specs=pl.BlockSpec((BN, 1), lambda i: (i, 0)),
        out_shape=jax.ShapeDtypeStruct((N, 1), jnp.float32),
    )(x, Wl.reshape(1, F), bl.reshape(1, 1))


def _gather_relu_sc(A, B, src, dst):
    E = src.shape[0]
    H = A.shape[1]
    Epc = E // _NW
    C = 128
    nfull = Epc // C
    Ct = Epc - nfull * C
    mesh = plsc.VectorSubcoreMesh(core_axis_name="c", subcore_axis_name="s")

    @functools.partial(
        pl.kernel, mesh=mesh,
        compiler_params=pltpu.CompilerParams(use_tc_tiling_on_sc=False, needs_layout_passes=False),
        out_type=jax.ShapeDtypeStruct((E, H), jnp.float32),
        scratch_types=[
            pltpu.VMEM((C,), jnp.int32),
            pltpu.VMEM((C,), jnp.int32),
            pltpu.VMEM((C, H), jnp.float32),
            pltpu.VMEM((C, H), jnp.float32),
            pltpu.SemaphoreType.DMA,
            pltpu.SemaphoreType.DMA,
        ],
    )
    def k(a_hbm, b_hbm, src_hbm, dst_hbm, out_hbm, sidx, didx, arows, brows,
          sema, semb):
        w = lax.axis_index("s") * 2 + lax.axis_index("c")
        base = w * Epc
        zero = jnp.zeros((16,), jnp.int32)
        for j in range(C // 16):
            sidx[pl.ds(j * 16, 16)] = zero
            didx[pl.ds(j * 16, 16)] = zero

        def do_chunk(e0, n):
            pltpu.sync_copy(src_hbm.at[pl.ds(e0, n)], sidx.at[pl.ds(0, n)])
            pltpu.sync_copy(dst_hbm.at[pl.ds(e0, n)], didx.at[pl.ds(0, n)])
            ca = pltpu.async_copy(a_hbm.at[sidx], arows, sema)
            cb = pltpu.async_copy(b_hbm.at[didx], brows, semb)
            ca.wait()
            cb.wait()

            def rowfn(r, carry):
                for c4 in range(H // 16):
                    sl = pl.ds(c4 * 16, 16)
                    arows[r, sl] = jnp.maximum(arows[r, sl] - brows[r, sl],
                                               0.0)
                return carry

            lax.fori_loop(0, n, rowfn, 0)
            pltpu.sync_copy(arows.at[pl.ds(0, n)],
                            out_hbm.at[pl.ds(e0, n)])

        def chunk_body(i, carry):
            do_chunk(base + i * C, C)
            return carry

        lax.fori_loop(0, nfull, chunk_body, 0)
        if Ct:
            do_chunk(base + nfull * C, Ct)

    return k(A, B, src, dst)


_NEG = float("-inf")


def _segmax_sc(h2big, dst_pad, E, F, N_out):
    NW = _NW
    Epc = E // NW
    C = 512
    W = 512
    mesh = plsc.VectorSubcoreMesh(core_axis_name="c", subcore_axis_name="s")
    h2flat = h2big.reshape(-1)

    @functools.partial(
        pl.kernel, mesh=mesh,
        compiler_params=pltpu.CompilerParams(use_tc_tiling_on_sc=False, needs_layout_passes=False),
        out_type=jax.ShapeDtypeStruct((N_out * F,), jnp.float32),
        scratch_types=[
            pltpu.VMEM((C * F,), jnp.float32),
            pltpu.VMEM((C,), jnp.int32),
            pltpu.VMEM((16,), jnp.int32),
            pltpu.VMEM((W * F,), jnp.float32),
        ],
    )
    def k(h2_hbm, dst_hbm, out_hbm, h2b, dstb, d16, buf):
        w = lax.axis_index("s") * 2 + lax.axis_index("c")
        e0 = w * Epc
        e1 = e0 + Epc
        iota = lax.iota(jnp.int32, 16)
        neg = jnp.full((16,), _NEG, jnp.float32)

        def lane_of(v, i):
            return jnp.max(jnp.where(iota == i, v, -(2**31) + 1))

        def read_dst(q):
            qf = pl.multiple_of((q >> 4) << 4, 16)
            pltpu.sync_copy(dst_hbm.at[pl.ds(qf, 16)], d16)
            return lane_of(d16[...], q - qf)

        def search(q0, val):
            def cond(st):
                return jnp.logical_not(st[1])

            def body(st):
                q, found, res = st
                qf = pl.multiple_of((q >> 4) << 4, 16)
                pltpu.sync_copy(dst_hbm.at[pl.ds(qf, 16)], d16)
                dv = d16[...]
                pos = qf + iota
                m = (pos >= q) & ((dv != val) | (pos >= E))
                anym = jnp.any(m)
                ffs = jnp.max(plsc.all_reduce_ffs(m))
                res2 = jnp.where(found | ~anym, res, qf + ffs)
                return (qf + 16, found | anym, res2)

            return lax.while_loop(cond, body, (q0, jnp.bool_(False), q0))[2]

        def memset_buf():
            def mb(i, c):
                buf[pl.ds(i * 16, 16)] = neg
                return c
            lax.fori_loop(0, W * F // 16, mb, 0)

        def flush(wb):
            off = pl.multiple_of(wb * F, 8)
            pltpu.sync_copy(buf, out_hbm.at[pl.ds(off, W * F)])
            memset_buf()

        def shift_down(v, s):
            return v.at[jnp.maximum(iota - s, 0)].get(mode="promise_in_bounds")

        dprev = jnp.where(w > 0, read_dst(jnp.maximum(e0 - 1, 0)), -1)
        dlast = read_dst(e1 - 1)
        own_end = jnp.where(w == NW - 1, N_out - 1, dlast)
        e_start = search(e0, dprev)
        e_end = search(e1, dlast)
        r = e_start - ((e_start >> 3) << 3)

        memset_buf()

        def process_group(g, cb, wbase):
            lidx = (g - cb) + iota
            dvr = plsc.load_gather(dstb, [lidx])
            lanemask = (g + iota) < e_end
            dveq = jnp.where(lanemask, dvr, -2 - iota)
            up = dveq.at[jnp.minimum(iota + 1, 15)].get(
                mode="promise_in_bounds")
            fin = (dveq != up) | (iota == 15)
            eqs = []
            for s in (1, 2, 4, 8):
                eqs.append((dveq == shift_down(dveq, s)) & (iota >= s))
            lF = lidx * F

            def wcond(st):
                return jnp.logical_not(jnp.all(st[0]))

            def wbody(st):
                done, wb = st
                m_in = (~done) & (dvr < wb + W)
                scat = m_in & fin
                offb = (dvr - wb) * F
                for kf in range(F):
                    x = plsc.load_gather(h2b, [lF + kf])
                    for si, s in enumerate((1, 2, 4, 8)):
                        x = jnp.maximum(
                            x, jnp.where(eqs[si], shift_down(x, s), neg))
                    cur = plsc.load_gather(buf, [offb + kf], mask=scat)
                    plsc.store_scatter(buf, [offb + kf],
                                       jnp.maximum(x, cur), mask=scat)
                done2 = done | m_in
                rem = jnp.logical_not(jnp.all(done2))

                @pl.when(rem)
                def _():
                    flush(wb)

                return (done2, jnp.where(rem, wb + W, wb))

            st = lax.while_loop(wcond, wbody,
                                (jnp.logical_not(lanemask), wbase))
            return st[1]

        def ocond(st):
            return st[0] < e_end

        def obody(st):
            g, wb = st
            cb = pl.multiple_of(g - r, 8)
            pltpu.sync_copy(dst_hbm.at[pl.ds(cb, C)], dstb)
            pltpu.sync_copy(h2_hbm.at[pl.ds(pl.multiple_of(cb * F, 8), C * F)], h2b)

            def icond(st2):
                return (st2[0] + 16 <= cb + C) & (st2[0] < e_end)

            def ibody(st2):
                g2, wb2 = st2
                return (g2 + 16, process_group(g2, cb, wb2))

            return lax.while_loop(icond, ibody, (g, wb))

        wb = lax.while_loop(ocond, obody, (e_start, dprev + 1))[1]

        wb = lax.while_loop(
            lambda v: v + W <= own_end + 1,
            lambda v: (flush(v), v + W)[1], wb)

        def pbody(st):
            wb2, o = st
            pltpu.sync_copy(buf.at[pl.ds(pl.multiple_of(o, 8), 8 * F)],
                            out_hbm.at[pl.ds(pl.multiple_of(wb2 * F, 8),
                                             8 * F)])
            return (wb2 + 8, o + 8 * F)

        st = lax.while_loop(lambda s2: s2[0] + 8 <= own_end + 1, pbody,
                            (wb, 0))

        def qbody(st2):
            wb2, o = st2
            pltpu.sync_copy(buf.at[pl.ds(pl.multiple_of(o, 8), F)],
                            out_hbm.at[pl.ds(pl.multiple_of(wb2 * F, 8), F)])
            return (wb2 + 1, o + F)

        lax.while_loop(lambda s2: s2[0] <= own_end, qbody, st)

    return k(h2flat, dst_pad)


def _celu_fin(x):
    x = jnp.where(jnp.isfinite(x), x, 0.0)
    return jnp.where(x > 0, x, (jnp.exp(x) - 1.0))


def _layer(x_act, pos_in, pos_out, src, dst, Wa, ba, Wb, bb, act):
    F = Wa.shape[0] - 3
    H = Wa.shape[1]
    Fo = Wb.shape[1]
    pad = (-H) % 16
    Wa_p = Wa
    Wb_p = Wb
    if pad:
        Wa_p = jnp.pad(Wa, ((0, 0), (0, pad)))
        Wb_p = jnp.pad(Wb, ((0, pad), (0, 0)))
        ba = jnp.pad(ba, (0, pad))
    A = _prep_A(x_act, pos_in, Wa_p[:F], Wa_p[F:], ba, act)
    B = _prep_B(pos_out, Wa_p[F:])
    G = _gather_relu_sc(A, B, src, dst)
    USE_SC_SEGMAX = True
    PAD = 1024
    h2 = _edge_mlp(G, Wb_p, bb, PAD)
    N_out = pos_out.shape[0]
    if USE_SC_SEGMAX:
        dst_pad = jnp.pad(dst, (0, PAD))
        seg = _segmax_sc(h2, dst_pad, src.shape[0], Fo, N_out)
        return seg.reshape(N_out, Fo)
    return jax.ops.segment_max(h2[:src.shape[0]], dst, num_segments=N_out)


def kernel(z_mask, pos1, pos2, pos3, pos4, gidx1, gidx2, gidx3, gidx4,
           e1_src, e1_dst, e2_src, e2_dst, e3_src, e3_dst,
           W1a, b1a, W1b, b1b, W2a, b2a, W2b, b2b, W3a, b3a, W3b, b3b, Wl, bl):
    s1 = _layer(z_mask, pos4, pos3, e1_src, e1_dst, W1a, b1a, W1b, b1b, False)
    s2 = _layer(s1, pos3, pos2, e2_src, e2_dst, W2a, b2a, W2b, b2b, True)
    s3 = _layer(s2, pos2, pos1, e3_src, e3_dst, W3a, b3a, W3b, b3b, True)
    return _final(s3, Wl, bl)


# speedup vs baseline: 5.6290x; 1.1247x over previous
import functools

import jax
import jax.numpy as jnp
from jax import lax
from jax.experimental import pallas as pl
from jax.experimental.pallas import tpu as pltpu
from jax.experimental.pallas import tpu_sc as plsc

_NW = 32


def _prep_block(x_ref, p_ref, wt_ref, wp_ref, b_ref, o_ref, *, act):
    x = x_ref[...]
    if act:
        x = jnp.where(jnp.isfinite(x), x, 0.0)
        x = jnp.where(x > 0, x, (jnp.exp(x) - 1.0))
    o_ref[...] = (jnp.dot(x, wt_ref[...], preferred_element_type=jnp.float32)
                  + jnp.dot(p_ref[...], wp_ref[...],
                            preferred_element_type=jnp.float32)
                  + b_ref[...])


def _pick_bn(N):
    for bn in (4000, 2000, 1000, 200, 40):
        if N % bn == 0:
            return bn
    return N


def _prep_A(x, pos, Wt, Wp, ba, act):
    N, F = x.shape
    H = Wt.shape[1]
    BN = _pick_bn(N)
    return pl.pallas_call(
        functools.partial(_prep_block, act=act),
        grid=(N // BN,),
        in_specs=[
            pl.BlockSpec((BN, F), lambda i: (i, 0)),
            pl.BlockSpec((BN, 3), lambda i: (i, 0)),
            pl.BlockSpec((F, H), lambda i: (0, 0)),
            pl.BlockSpec((3, H), lambda i: (0, 0)),
            pl.BlockSpec((1, H), lambda i: (0, 0)),
        ],
        out_specs=pl.BlockSpec((BN, H), lambda i: (i, 0)),
        out_shape=jax.ShapeDtypeStruct((N, H), jnp.float32),
    )(x, pos, Wt, Wp, ba.reshape(1, H))


def _prep_B_block(p_ref, wp_ref, o_ref):
    o_ref[...] = jnp.dot(p_ref[...], wp_ref[...],
                         preferred_element_type=jnp.float32)


def _prep_B(pos, Wp):
    N = pos.shape[0]
    H = Wp.shape[1]
    BN = _pick_bn(N)
    return pl.pallas_call(
        _prep_B_block,
        grid=(N // BN,),
        in_specs=[
            pl.BlockSpec((BN, 3), lambda i: (i, 0)),
            pl.BlockSpec((3, H), lambda i: (0, 0)),
        ],
        out_specs=pl.BlockSpec((BN, H), lambda i: (i, 0)),
        out_shape=jax.ShapeDtypeStruct((N, H), jnp.float32),
    )(pos, Wp)


def _edge_mlp_block(g_ref, w_ref, b_ref, o_ref):
    o_ref[...] = (jnp.dot(g_ref[...], w_ref[...],
                          preferred_element_type=jnp.float32) + b_ref[...])


def _edge_mlp(G, Wb, bb, pad_rows, BE=6400):
    E, H = G.shape
    F = Wb.shape[1]
    return pl.pallas_call(
        _edge_mlp_block,
        grid=(E // BE,),
        in_specs=[
            pl.BlockSpec((BE, H), lambda i: (i, 0)),
            pl.BlockSpec((H, F), lambda i: (0, 0)),
            pl.BlockSpec((1, F), lambda i: (0, 0)),
        ],
        out_specs=pl.BlockSpec((BE, F), lambda i: (i, 0)),
        out_shape=jax.ShapeDtypeStruct((E + pad_rows, F), jnp.float32),
    )(G, Wb, bb.reshape(1, F))


def _final_block(x_ref, w_ref, b_ref, o_ref):
    x = x_ref[...]
    x = jnp.where(jnp.isfinite(x), x, 0.0)
    x = jnp.where(x > 0, x, (jnp.exp(x) - 1.0))
    y = jnp.sum(x * w_ref[...], axis=1, keepdims=True) + b_ref[0, 0]
    o_ref[...] = jnp.minimum(y, 0.0) - jnp.log(1.0 + jnp.exp(-jnp.abs(y)))


def _final(x, Wl, bl, BN=25000):
    N, F = x.shape
    return pl.pallas_call(
        _final_block,
        grid=(N // BN,),
        in_specs=[
            pl.BlockSpec((BN, F), lambda i: (i, 0)),
            pl.BlockSpec((1, F), lambda i: (0, 0)),
            pl.BlockSpec((1, 1), lambda i: (0, 0)),
        ],
        out_specs=pl.BlockSpec((BN, 1), lambda i: (i, 0)),
        out_shape=jax.ShapeDtypeStruct((N, 1), jnp.float32),
    )(x, Wl.reshape(1, F), bl.reshape(1, 1))


def _gather_relu_sc(A, B, src, dst):
    E = src.shape[0]
    H = A.shape[1]
    Epc = E // _NW
    C = 128
    nfull = Epc // C
    Ct = Epc - nfull * C
    mesh = plsc.VectorSubcoreMesh(core_axis_name="c", subcore_axis_name="s")

    @functools.partial(
        pl.kernel, mesh=mesh,
        compiler_params=pltpu.CompilerParams(use_tc_tiling_on_sc=False, needs_layout_passes=False),
        out_type=jax.ShapeDtypeStruct((E, H), jnp.float32),
        scratch_types=[
            pltpu.VMEM((2, C), jnp.int32),
            pltpu.VMEM((2, C), jnp.int32),
            pltpu.VMEM((2, C, H), jnp.float32),
            pltpu.VMEM((2, C, H), jnp.float32),
            pltpu.SemaphoreType.DMA,
            pltpu.SemaphoreType.DMA,
            pltpu.SemaphoreType.DMA,
        ],
    )
    def k(a_hbm, b_hbm, src_hbm, dst_hbm, out_hbm, sidx, didx, arows, brows,
          sema, semb, semo):
        w = lax.axis_index("s") * 2 + lax.axis_index("c")
        base = w * Epc
        zero = jnp.zeros((16,), jnp.int32)
        for j in range(C // 16):
            sidx[0, pl.ds(j * 16, 16)] = zero
            didx[0, pl.ds(j * 16, 16)] = zero
            sidx[1, pl.ds(j * 16, 16)] = zero
            didx[1, pl.ds(j * 16, 16)] = zero

        def fire(i, p):
            e0 = base + i * C
            pltpu.sync_copy(src_hbm.at[pl.ds(e0, C)], sidx.at[p])
            pltpu.sync_copy(dst_hbm.at[pl.ds(e0, C)], didx.at[p])
            pltpu.async_copy(a_hbm.at[sidx.at[p]], arows.at[p], sema)
            pltpu.async_copy(b_hbm.at[didx.at[p]], brows.at[p], semb)

        def compute(p, n):
            def rowfn(r4, carry):
                for rr in range(4):
                    r = r4 * 4 + rr
                    for c4 in range(H // 16):
                        sl = pl.ds(c4 * 16, 16)
                        arows[p, r, sl] = jnp.maximum(
                            arows[p, r, sl] - brows[p, r, sl], 0.0)
                return carry

            lax.fori_loop(0, n // 4, rowfn, 0)

        def wait_gathers(p):
            pltpu.make_async_copy(a_hbm.at[sidx.at[p]], arows.at[p],
                                  sema).wait()
            pltpu.make_async_copy(b_hbm.at[didx.at[p]], brows.at[p],
                                  semb).wait()

        def wait_out(i, p):
            e0 = base + i * C
            pltpu.make_async_copy(arows.at[p],
                                  out_hbm.at[pl.ds(e0, C)], semo).wait()

        fire(0, 0)

        def chunk_body(i, carry):
            p = lax.rem(i, 2)
            q = 1 - p

            @pl.when(i + 1 < nfull)
            def _():
                @pl.when(i >= 1)
                def _():
                    wait_out(i - 1, q)

                fire(i + 1, q)

            wait_gathers(p)
            compute(p, C)
            pltpu.async_copy(arows.at[p],
                             out_hbm.at[pl.ds(base + i * C, C)], semo)
            return carry

        lax.fori_loop(0, nfull, chunk_body, 0)
        wait_out(nfull - 2, lax.rem(nfull - 2, 2))
        wait_out(nfull - 1, lax.rem(nfull - 1, 2))

        if Ct:
            e0 = base + nfull * C
            pltpu.sync_copy(src_hbm.at[pl.ds(e0, Ct)],
                            sidx.at[0, pl.ds(0, Ct)])
            pltpu.sync_copy(dst_hbm.at[pl.ds(e0, Ct)],
                            didx.at[0, pl.ds(0, Ct)])
            ca = pltpu.async_copy(a_hbm.at[sidx.at[0]], arows.at[0], sema)
            cb = pltpu.async_copy(b_hbm.at[didx.at[0]], brows.at[0], semb)
            ca.wait()
            cb.wait()
            compute(0, Ct)
            pltpu.sync_copy(arows.at[0, pl.ds(0, Ct)],
                            out_hbm.at[pl.ds(e0, Ct)])

    return k(A, B, src, dst)


_NEG = float("-inf")


def _segmax_sc(h2big, dst_pad, E, F, N_out):
    NW = _NW
    Epc = E // NW
    C = 512
    W = 512
    mesh = plsc.VectorSubcoreMesh(core_axis_name="c", subcore_axis_name="s")
    h2flat = h2big.reshape(-1)

    @functools.partial(
        pl.kernel, mesh=mesh,
        compiler_params=pltpu.CompilerParams(use_tc_tiling_on_sc=False, needs_layout_passes=False),
        out_type=jax.ShapeDtypeStruct((N_out * F,), jnp.float32),
        scratch_types=[
            pltpu.VMEM((C * F,), jnp.float32),
            pltpu.VMEM((C,), jnp.int32),
            pltpu.VMEM((16,), jnp.int32),
            pltpu.VMEM((W * F,), jnp.float32),
        ],
    )
    def k(h2_hbm, dst_hbm, out_hbm, h2b, dstb, d16, buf):
        w = lax.axis_index("s") * 2 + lax.axis_index("c")
        e0 = w * Epc
        e1 = e0 + Epc
        iota = lax.iota(jnp.int32, 16)
        neg = jnp.full((16,), _NEG, jnp.float32)

        def lane_of(v, i):
            return jnp.max(jnp.where(iota == i, v, -(2**31) + 1))

        def read_dst(q):
            qf = pl.multiple_of((q >> 4) << 4, 16)
            pltpu.sync_copy(dst_hbm.at[pl.ds(qf, 16)], d16)
            return lane_of(d16[...], q - qf)

        def search(q0, val):
            def cond(st):
                return jnp.logical_not(st[1])

            def body(st):
                q, found, res = st
                qf = pl.multiple_of((q >> 4) << 4, 16)
                pltpu.sync_copy(dst_hbm.at[pl.ds(qf, 16)], d16)
                dv = d16[...]
                pos = qf + iota
                m = (pos >= q) & ((dv != val) | (pos >= E))
                anym = jnp.any(m)
                ffs = jnp.max(plsc.all_reduce_ffs(m))
                res2 = jnp.where(found | ~anym, res, qf + ffs)
                return (qf + 16, found | anym, res2)

            return lax.while_loop(cond, body, (q0, jnp.bool_(False), q0))[2]

        def memset_buf():
            def mb(i, c):
                buf[pl.ds(i * 16, 16)] = neg
                return c
            lax.fori_loop(0, W * F // 16, mb, 0)

        def flush(wb):
            off = pl.multiple_of(wb * F, 8)
            pltpu.sync_copy(buf, out_hbm.at[pl.ds(off, W * F)])
            memset_buf()

        def shift_down(v, s):
            return v.at[jnp.maximum(iota - s, 0)].get(mode="promise_in_bounds")

        dprev = jnp.where(w > 0, read_dst(jnp.maximum(e0 - 1, 0)), -1)
        dlast = read_dst(e1 - 1)
        own_end = jnp.where(w == NW - 1, N_out - 1, dlast)
        e_start = search(e0, dprev)
        e_end = search(e1, dlast)
        r = e_start - ((e_start >> 3) << 3)

        memset_buf()

        def process_group(g, cb, wbase):
            lidx = (g - cb) + iota
            dvr = plsc.load_gather(dstb, [lidx])
            lanemask = (g + iota) < e_end
            dveq = jnp.where(lanemask, dvr, -2 - iota)
            up = dveq.at[jnp.minimum(iota + 1, 15)].get(
                mode="promise_in_bounds")
            fin = (dveq != up) | (iota == 15)
            eqs = []
            for s in (1, 2, 4, 8):
                eqs.append((dveq == shift_down(dveq, s)) & (iota >= s))
            lF = lidx * F

            def wcond(st):
                return jnp.logical_not(jnp.all(st[0]))

            def wbody(st):
                done, wb = st
                m_in = (~done) & (dvr < wb + W)
                scat = m_in & fin
                offb = (dvr - wb) * F
                for kf in range(F):
                    x = plsc.load_gather(h2b, [lF + kf])
                    for si, s in enumerate((1, 2, 4, 8)):
                        x = jnp.maximum(
                            x, jnp.where(eqs[si], shift_down(x, s), neg))
                    cur = plsc.load_gather(buf, [offb + kf], mask=scat)
                    plsc.store_scatter(buf, [offb + kf],
                                       jnp.maximum(x, cur), mask=scat)
                done2 = done | m_in
                rem = jnp.logical_not(jnp.all(done2))

                @pl.when(rem)
                def _():
                    flush(wb)

                return (done2, jnp.where(rem, wb + W, wb))

            st = lax.while_loop(wcond, wbody,
                                (jnp.logical_not(lanemask), wbase))
            return st[1]

        def ocond(st):
            return st[0] < e_end

        def obody(st):
            g, wb = st
            cb = pl.multiple_of(g - r, 8)
            pltpu.sync_copy(dst_hbm.at[pl.ds(cb, C)], dstb)
            pltpu.sync_copy(h2_hbm.at[pl.ds(pl.multiple_of(cb * F, 8), C * F)], h2b)

            def icond(st2):
                return (st2[0] + 16 <= cb + C) & (st2[0] < e_end)

            def ibody(st2):
                g2, wb2 = st2
                return (g2 + 16, process_group(g2, cb, wb2))

            return lax.while_loop(icond, ibody, (g, wb))

        wb = lax.while_loop(ocond, obody, (e_start, dprev + 1))[1]

        wb = lax.while_loop(
            lambda v: v + W <= own_end + 1,
            lambda v: (flush(v), v + W)[1], wb)

        def pbody(st):
            wb2, o = st
            pltpu.sync_copy(buf.at[pl.ds(pl.multiple_of(o, 8), 8 * F)],
                            out_hbm.at[pl.ds(pl.multiple_of(wb2 * F, 8),
                                             8 * F)])
            return (wb2 + 8, o + 8 * F)

        st = lax.while_loop(lambda s2: s2[0] + 8 <= own_end + 1, pbody,
                            (wb, 0))

        def qbody(st2):
            wb2, o = st2
            pltpu.sync_copy(buf.at[pl.ds(pl.multiple_of(o, 8), F)],
                            out_hbm.at[pl.ds(pl.multiple_of(wb2 * F, 8), F)])
            return (wb2 + 1, o + F)

        lax.while_loop(lambda s2: s2[0] <= own_end, qbody, st)

    return k(h2flat, dst_pad)


def _celu_fin(x):
    x = jnp.where(jnp.isfinite(x), x, 0.0)
    return jnp.where(x > 0, x, (jnp.exp(x) - 1.0))


def _layer(x_act, pos_in, pos_out, src, dst, Wa, ba, Wb, bb, act):
    F = Wa.shape[0] - 3
    H = Wa.shape[1]
    Fo = Wb.shape[1]
    pad = (-H) % 16
    Wa_p = Wa
    Wb_p = Wb
    if pad:
        Wa_p = jnp.pad(Wa, ((0, 0), (0, pad)))
        Wb_p = jnp.pad(Wb, ((0, pad), (0, 0)))
        ba = jnp.pad(ba, (0, pad))
    A = _prep_A(x_act, pos_in, Wa_p[:F], Wa_p[F:], ba, act)
    B = _prep_B(pos_out, Wa_p[F:])
    G = _gather_relu_sc(A, B, src, dst)
    USE_SC_SEGMAX = True
    PAD = 1024
    h2 = _edge_mlp(G, Wb_p, bb, PAD)
    N_out = pos_out.shape[0]
    if USE_SC_SEGMAX:
        dst_pad = jnp.pad(dst, (0, PAD))
        seg = _segmax_sc(h2, dst_pad, src.shape[0], Fo, N_out)
        return seg.reshape(N_out, Fo)
    return jax.ops.segment_max(h2[:src.shape[0]], dst, num_segments=N_out)


def kernel(z_mask, pos1, pos2, pos3, pos4, gidx1, gidx2, gidx3, gidx4,
           e1_src, e1_dst, e2_src, e2_dst, e3_src, e3_dst,
           W1a, b1a, W1b, b1b, W2a, b2a, W2b, b2b, W3a, b3a, W3b, b3b, Wl, bl):
    s1 = _layer(z_mask, pos4, pos3, e1_src, e1_dst, W1a, b1a, W1b, b1b, False)
    s2 = _layer(s1, pos3, pos2, e2_src, e2_dst, W2a, b2a, W2b, b2b, True)
    s3 = _layer(s2, pos2, pos1, e3_src, e3_dst, W3a, b3a, W3b, b3b, True)
    return _final(s3, Wl, bl)
